# baseline (speedup 1.0000x reference)
import functools

import jax
import jax.numpy as jnp
import numpy as np
from jax import lax
from jax.experimental import pallas as pl
from jax.experimental.pallas import tpu as pltpu
from jax.experimental.pallas import tpu_sc as plsc

_ITEM_Z = np.array(
    [1, 1, 1, 0, 1, 1, 1, 1, 1, 1, 0, 1, 1, 1, 1, 1,
     1, 0, 1, 1, 1, 1, 1, 1, 0, 1, 1, 1, 1, 1, 1, 1],
    dtype=np.int64,
)
_INPUTS_PER_ITEM = 64
_N_ITEMS = _ITEM_Z.size
_OUT_FEATURES = _N_ITEMS * _INPUTS_PER_ITEM
_N_ACTIVE = int(_ITEM_Z.sum())
_ZERO_OUT_IDX = np.array([0, 63, 100, 511, 1024, 1500, 2047], dtype=np.int64)

_ZERO_SLOT = _N_ACTIVE
_WP_PAD = 32

_rank_of_item = np.cumsum(_ITEM_Z) - 1
_IDX_MAP = np.where(
    np.repeat(_ITEM_Z, _INPUTS_PER_ITEM) == 1,
    np.repeat(_rank_of_item, _INPUTS_PER_ITEM),
    _ZERO_SLOT,
).astype(np.int32)
_IDX_MAP[_ZERO_OUT_IDX] = _ZERO_SLOT

_ROW_BLOCK = 512

_SC_INFO = plsc.get_sparse_core_info()
_NC, _NS, _L = 1, _SC_INFO.num_subcores, _SC_INFO.num_lanes
_NW = _NC * _NS
_PER_W = _OUT_FEATURES // _NW
_VREGS_PER_W = _PER_W // _L


def _sc_scatter_body(wp_hbm, idx_hbm, w_hbm, wp_v, idx_v, w_v):
    wid = lax.axis_index("s") * _NC + lax.axis_index("c")
    base = wid * _PER_W
    pltpu.sync_copy(wp_hbm, wp_v)
    pltpu.sync_copy(idx_hbm.at[pl.ds(base, _PER_W)], idx_v)
    for j in range(_VREGS_PER_W):
        idxs = idx_v[pl.ds(j * _L, _L)]
        w_v[pl.ds(j * _L, _L)] = plsc.load_gather(wp_v, [idxs])
    pltpu.sync_copy(w_v, w_hbm.at[pl.ds(base, _PER_W)])


def _sc_scatter(wp_padded, idx_map):
    mesh = plsc.VectorSubcoreMesh(
        core_axis_name="c", subcore_axis_name="s", num_cores=_NC
    )
    return pl.kernel(
        _sc_scatter_body,
        mesh=mesh,
        out_type=jax.ShapeDtypeStruct((_OUT_FEATURES,), jnp.float32),
        scratch_types=[
            pltpu.VMEM((_WP_PAD,), jnp.float32),
            pltpu.VMEM((_PER_W,), jnp.int32),
            pltpu.VMEM((_PER_W,), jnp.float32),
        ],
        compiler_params=pltpu.CompilerParams(needs_layout_passes=False),
    )(wp_padded, idx_map)


def _mul_body(w_ref, x_ref, o_ref):
    o_ref[...] = x_ref[...] * w_ref[...]


@jax.jit
def kernel(x, weight_param):
    batch, feats = x.shape
    wp_padded = jnp.concatenate(
        [weight_param, jnp.zeros((_WP_PAD - _N_ACTIVE,), jnp.float32)]
    )
    w = _sc_scatter(wp_padded, jnp.asarray(_IDX_MAP)).reshape(1, feats)
    grid = (batch // _ROW_BLOCK,)
    return pl.pallas_call(
        _mul_body,
        grid=grid,
        in_specs=[
            pl.BlockSpec((1, feats), lambda i: (0, 0)),
            pl.BlockSpec((_ROW_BLOCK, feats), lambda i: (i, 0)),
        ],
        out_specs=pl.BlockSpec((_ROW_BLOCK, feats), lambda i: (i, 0)),
        out_shape=jax.ShapeDtypeStruct((batch, feats), x.dtype),
    )(w, x)


# speedup vs baseline: 1.0210x; 1.0210x over previous
import functools

import jax
import jax.numpy as jnp
import numpy as np
from jax import lax
from jax.experimental import pallas as pl
from jax.experimental.pallas import tpu as pltpu
from jax.experimental.pallas import tpu_sc as plsc

_ITEM_Z = np.array(
    [1, 1, 1, 0, 1, 1, 1, 1, 1, 1, 0, 1, 1, 1, 1, 1,
     1, 0, 1, 1, 1, 1, 1, 1, 0, 1, 1, 1, 1, 1, 1, 1],
    dtype=np.int64,
)
_INPUTS_PER_ITEM = 64
_N_ITEMS = _ITEM_Z.size
_OUT_FEATURES = _N_ITEMS * _INPUTS_PER_ITEM
_N_ACTIVE = int(_ITEM_Z.sum())
_ZERO_OUT_IDX = np.array([0, 63, 100, 511, 1024, 1500, 2047], dtype=np.int64)

_ZERO_SLOT = _N_ACTIVE
_WP_PAD = 32

_rank_of_item = np.cumsum(_ITEM_Z) - 1
_IDX_MAP = np.where(
    np.repeat(_ITEM_Z, _INPUTS_PER_ITEM) == 1,
    np.repeat(_rank_of_item, _INPUTS_PER_ITEM),
    _ZERO_SLOT,
).astype(np.int32)
_IDX_MAP[_ZERO_OUT_IDX] = _ZERO_SLOT

_ROW_BLOCK = 1024

_SC_INFO = plsc.get_sparse_core_info()
_NC, _NS, _L = 1, _SC_INFO.num_subcores, _SC_INFO.num_lanes
_NW = _NC * _NS
_PER_W = _OUT_FEATURES // _NW
_VREGS_PER_W = _PER_W // _L


def _sc_scatter_body(wp_hbm, idx_hbm, w_hbm, wp_v, idx_v, w_v):
    wid = lax.axis_index("s") * _NC + lax.axis_index("c")
    base = wid * _PER_W
    pltpu.sync_copy(wp_hbm, wp_v.at[pl.ds(0, _N_ACTIVE)])
    pltpu.sync_copy(idx_hbm.at[pl.ds(base, _PER_W)], idx_v)
    lane = lax.iota(jnp.int32, _L)
    tail = wp_v[pl.ds(_L, _L)]
    wp_v[pl.ds(_L, _L)] = jnp.where(lane < _N_ACTIVE - _L, tail, 0.0)
    for j in range(_VREGS_PER_W):
        idxs = idx_v[pl.ds(j * _L, _L)]
        w_v[pl.ds(j * _L, _L)] = plsc.load_gather(wp_v, [idxs])
    pltpu.sync_copy(w_v, w_hbm.at[pl.ds(base, _PER_W)])


def _sc_scatter(wp_padded, idx_map):
    mesh = plsc.VectorSubcoreMesh(
        core_axis_name="c", subcore_axis_name="s", num_cores=_NC
    )
    return pl.kernel(
        _sc_scatter_body,
        mesh=mesh,
        out_type=jax.ShapeDtypeStruct((_OUT_FEATURES,), jnp.float32),
        scratch_types=[
            pltpu.VMEM((_WP_PAD,), jnp.float32),
            pltpu.VMEM((_PER_W,), jnp.int32),
            pltpu.VMEM((_PER_W,), jnp.float32),
        ],
        compiler_params=pltpu.CompilerParams(needs_layout_passes=False),
    )(wp_padded, idx_map)


def _mul_body(w_ref, x_ref, o_ref):
    o_ref[...] = x_ref[...] * w_ref[...]


@jax.jit
def kernel(x, weight_param):
    batch, feats = x.shape
    w = _sc_scatter(weight_param, jnp.asarray(_IDX_MAP)).reshape(1, feats)
    grid = (batch // _ROW_BLOCK,)
    return pl.pallas_call(
        _mul_body,
        grid=grid,
        in_specs=[
            pl.BlockSpec((1, feats), lambda i: (0, 0)),
            pl.BlockSpec((_ROW_BLOCK, feats), lambda i: (i, 0)),
        ],
        out_specs=pl.BlockSpec((_ROW_BLOCK, feats), lambda i: (i, 0)),
        out_shape=jax.ShapeDtypeStruct((batch, feats), x.dtype),
    )(w, x)


# speedup vs baseline: 1.0568x; 1.0350x over previous
import functools

import jax
import jax.numpy as jnp
import numpy as np
from jax import lax
from jax.experimental import pallas as pl
from jax.experimental.pallas import tpu as pltpu
from jax.experimental.pallas import tpu_sc as plsc

_ITEM_Z = np.array(
    [1, 1, 1, 0, 1, 1, 1, 1, 1, 1, 0, 1, 1, 1, 1, 1,
     1, 0, 1, 1, 1, 1, 1, 1, 0, 1, 1, 1, 1, 1, 1, 1],
    dtype=np.int64,
)
_INPUTS_PER_ITEM = 64
_N_ITEMS = _ITEM_Z.size
_OUT_FEATURES = _N_ITEMS * _INPUTS_PER_ITEM
_N_ACTIVE = int(_ITEM_Z.sum())
_ZERO_OUT_IDX = np.array([0, 63, 100, 511, 1024, 1500, 2047], dtype=np.int64)

_ZERO_SLOT = _N_ACTIVE
_WP_PAD = 32

_rank_of_item = np.cumsum(_ITEM_Z) - 1
_IDX_MAP = np.where(
    np.repeat(_ITEM_Z, _INPUTS_PER_ITEM) == 1,
    np.repeat(_rank_of_item, _INPUTS_PER_ITEM),
    _ZERO_SLOT,
).astype(np.int32)
_IDX_MAP[_ZERO_OUT_IDX] = _ZERO_SLOT

_ROW_BLOCK = 1024

_SC_INFO = plsc.get_sparse_core_info()
_NC, _NS, _L = 1, _SC_INFO.num_subcores, _SC_INFO.num_lanes
_NW = _NC * _NS
_PER_W = _OUT_FEATURES // _NW
_VREGS_PER_W = _PER_W // _L


def _sc_scatter_body(wp_hbm, idx_hbm, w_hbm, wp_v, idx_v, w_v):
    wid = lax.axis_index("s") * _NC + lax.axis_index("c")
    base = wid * _PER_W
    pltpu.sync_copy(wp_hbm, wp_v.at[pl.ds(0, _N_ACTIVE)])
    pltpu.sync_copy(idx_hbm.at[pl.ds(base, _PER_W)], idx_v)
    lane = lax.iota(jnp.int32, _L)
    tail = wp_v[pl.ds(_L, _L)]
    wp_v[pl.ds(_L, _L)] = jnp.where(lane < _N_ACTIVE - _L, tail, 0.0)
    for j in range(_VREGS_PER_W):
        idxs = idx_v[pl.ds(j * _L, _L)]
        w_v[pl.ds(j * _L, _L)] = plsc.load_gather(wp_v, [idxs])
    pltpu.sync_copy(w_v, w_hbm.at[pl.ds(base, _PER_W)])


def _sc_scatter(wp_padded, idx_map):
    mesh = plsc.VectorSubcoreMesh(
        core_axis_name="c", subcore_axis_name="s", num_cores=_NC
    )
    return pl.kernel(
        _sc_scatter_body,
        mesh=mesh,
        out_type=jax.ShapeDtypeStruct((_OUT_FEATURES,), jnp.float32),
        scratch_types=[
            pltpu.VMEM((_WP_PAD,), jnp.float32),
            pltpu.VMEM((_PER_W,), jnp.int32),
            pltpu.VMEM((_PER_W,), jnp.float32),
        ],
        compiler_params=pltpu.CompilerParams(needs_layout_passes=False),
    )(wp_padded, idx_map)


_E = np.zeros((_N_ACTIVE, _OUT_FEATURES), dtype=np.float32)
for _i in range(_N_ITEMS):
    if _ITEM_Z[_i]:
        _E[_rank_of_item[_i], _i * _INPUTS_PER_ITEM:(_i + 1) * _INPUTS_PER_ITEM] = 1.0
_E[:, _ZERO_OUT_IDX] = 0.0

_HEAD_BLOCKS = 4
_HEAD_ROWS = _HEAD_BLOCKS * _ROW_BLOCK


def _mul_head_body(wp_ref, e_ref, x_ref, o_ref):
    w = jnp.dot(wp_ref[...], e_ref[...], preferred_element_type=jnp.float32)
    o_ref[...] = x_ref[...] * w


def _mul_tail_body(prev_ref, w_ref, x_ref, o_ref):
    del prev_ref
    o_ref[...] = x_ref[...] * w_ref[...]


@jax.jit
def kernel(x, weight_param):
    batch, feats = x.shape
    w = _sc_scatter(weight_param, jnp.asarray(_IDX_MAP)).reshape(1, feats)
    head = pl.pallas_call(
        _mul_head_body,
        grid=(_HEAD_BLOCKS,),
        in_specs=[
            pl.BlockSpec((1, _N_ACTIVE), lambda i: (0, 0)),
            pl.BlockSpec((_N_ACTIVE, feats), lambda i: (0, 0)),
            pl.BlockSpec((_ROW_BLOCK, feats), lambda i: (i, 0)),
        ],
        out_specs=pl.BlockSpec((_ROW_BLOCK, feats), lambda i: (i, 0)),
        out_shape=jax.ShapeDtypeStruct((batch, feats), x.dtype),
    )(weight_param.reshape(1, _N_ACTIVE), jnp.asarray(_E), x)
    tail_grid = (batch // _ROW_BLOCK - _HEAD_BLOCKS,)
    return pl.pallas_call(
        _mul_tail_body,
        grid=tail_grid,
        in_specs=[
            pl.BlockSpec(memory_space=pl.ANY),
            pl.BlockSpec((1, feats), lambda i: (0, 0)),
            pl.BlockSpec((_ROW_BLOCK, feats), lambda i: (i + _HEAD_BLOCKS, 0)),
        ],
        out_specs=pl.BlockSpec(
            (_ROW_BLOCK, feats), lambda i: (i + _HEAD_BLOCKS, 0)
        ),
        out_shape=jax.ShapeDtypeStruct((batch, feats), x.dtype),
        input_output_aliases={0: 0},
    )(head, w, x)
